# baseline (speedup 1.0000x reference)
import jax
import jax.numpy as jnp
from jax import lax
from jax.experimental import pallas as pl
from jax.experimental.pallas import tpu as pltpu
from jax.experimental.pallas import tpu_sc as plsc

_N = 50000
_E = 800000
_NH = 64
_NG = 64
_DOS = 51

_NC = 2
_NS = 16
_NW = _NC * _NS


def _dot(a, b):
    return jnp.dot(a, b, preferred_element_type=jnp.float32)


def _ln(h, g, bt):
    m = jnp.mean(h, axis=-1, keepdims=True)
    v = jnp.mean(h * h, axis=-1, keepdims=True) - m * m
    return (h - m) * lax.rsqrt(v + 1e-5) * g + bt


def _prelu(h, a):
    return jnp.where(h >= 0, h, a * h)


def _enc_node_body(x_ref, w1_ref, b1_ref, a_ref, w2_ref, b2_ref, o_ref):
    h = _dot(x_ref[...], w1_ref[...]) + b1_ref[...]
    h = _prelu(h, a_ref[0, 0])
    o_ref[...] = _dot(h, w2_ref[...]) + b2_ref[...]


def _enc_edge_body(ev_ref, es_ref, w1_ref, b1_ref, a_ref, w2_ref, b2_ref, o_ref):
    ev = ev_ref[...]
    el = jnp.sqrt(jnp.sum(ev * ev, axis=-1, keepdims=True))
    u = 2.0 * (el / 4.0 - 1.0)
    y = (1.0 - jnp.cos(jnp.pi * u)) / 2.0
    y = jnp.where(u > 0, 0.0, y)
    y = jnp.where(u < -1.0, 1.0, y)
    attr = y * es_ref[...]
    h = _dot(attr, w1_ref[...]) + b1_ref[...]
    h = _prelu(h, a_ref[0, 0])
    o_ref[...] = _dot(h, w2_ref[...]) + b2_ref[...]


def _edge_mlp_body(ar_ref, ac_ref, eh_ref, w1a_ref, w1b_ref, w1c_ref, b1_ref,
                   g_ref, bt_ref, a_ref, w2_ref, b2_ref, ne_ref, eo_ref):
    eh = eh_ref[...]
    h = (_dot(ar_ref[...], w1a_ref[...]) + _dot(ac_ref[...], w1b_ref[...])
         + _dot(eh, w1c_ref[...]) + b1_ref[...])
    h = _prelu(_ln(h, g_ref[...], bt_ref[...]), a_ref[0, 0])
    ne = _dot(h, w2_ref[...]) + b2_ref[...]
    ne_ref[...] = ne
    eo_ref[...] = eh + ne


def _edge_mlp1_body(ev_ref, es_ref, ar_ref, ac_ref,
                    ew1_ref, eb1_ref, ea_ref, ew2_ref, eb2_ref,
                    w1a_ref, w1b_ref, w1c_ref, b1_ref,
                    g_ref, bt_ref, a_ref, w2_ref, b2_ref, ne_ref, eo_ref):
    ev = ev_ref[...]
    el = jnp.sqrt(jnp.sum(ev * ev, axis=-1, keepdims=True))
    u = 2.0 * (el / 4.0 - 1.0)
    y = (1.0 - jnp.cos(jnp.pi * u)) / 2.0
    y = jnp.where(u > 0, 0.0, y)
    y = jnp.where(u < -1.0, 1.0, y)
    attr = y * es_ref[...]
    eh = _dot(attr, ew1_ref[...]) + eb1_ref[...]
    eh = _prelu(eh, ea_ref[0, 0])
    eh = _dot(eh, ew2_ref[...]) + eb2_ref[...]
    h = (_dot(ar_ref[...], w1a_ref[...]) + _dot(ac_ref[...], w1b_ref[...])
         + _dot(eh, w1c_ref[...]) + b1_ref[...])
    h = _prelu(_ln(h, g_ref[...], bt_ref[...]), a_ref[0, 0])
    ne = _dot(h, w2_ref[...]) + b2_ref[...]
    ne_ref[...] = ne
    eo_ref[...] = eh + ne


def _node_mlp_body(xh_ref, agg_ref, w1a_ref, w1b_ref, b1_ref,
                   g_ref, bt_ref, a_ref, w2_ref, b2_ref, o_ref):
    h = (_dot(xh_ref[...], w1a_ref[...]) + _dot(agg_ref[...], w1b_ref[...])
         + b1_ref[...])
    h = _prelu(_ln(h, g_ref[...], bt_ref[...]), a_ref[0, 0])
    o_ref[...] = xh_ref[...] + _dot(h, w2_ref[...]) + b2_ref[...]


def _out_body(xh_ref, b_ref, w1_ref, b1_ref, g_ref, bt_ref, a_ref, w2_ref,
              b2_ref, o_ref):
    i = pl.program_id(0)
    h = _dot(xh_ref[...], w1_ref[...]) + b1_ref[...]
    h = _prelu(_ln(h, g_ref[...], bt_ref[...]), a_ref[0, 0])
    dos = _dot(h, w2_ref[...]) + b2_ref[...]
    oh = (b_ref[...] == lax.broadcasted_iota(jnp.int32, (1, _NG), 1))
    oh = oh.astype(jnp.float32)
    contrib = lax.dot_general(oh, dos, (((0,), (0,)), ((), ())),
                              preferred_element_type=jnp.float32)

    @pl.when(i == 0)
    def _():
        o_ref[...] = jnp.zeros_like(o_ref)

    o_ref[...] += contrib


def _const2(shape):
    return pl.BlockSpec(shape, lambda i: (0, 0))


_SMEM_SPEC = pl.BlockSpec(memory_space=pltpu.SMEM)


def _enc_node(x, p):
    bn = 5000
    return pl.pallas_call(
        _enc_node_body,
        grid=(_N // bn,),
        in_specs=[
            pl.BlockSpec((bn, 128), lambda i: (i, 0)),
            _const2((128, _NH)), _const2((1, _NH)), _SMEM_SPEC,
            _const2((_NH, _NH)), _const2((1, _NH)),
        ],
        out_specs=pl.BlockSpec((bn, _NH), lambda i: (i, 0)),
        out_shape=jax.ShapeDtypeStruct((_N, _NH), jnp.float32),
    )(x, p["l1"]["w"], p["l1"]["b"].reshape(1, -1), p["a"].reshape(1, 1),
      p["l2"]["w"], p["l2"]["b"].reshape(1, -1))


def _enc_edge(ev, es, p):
    be = 8000
    return pl.pallas_call(
        _enc_edge_body,
        grid=(_E // be,),
        in_specs=[
            pl.BlockSpec((be, 3), lambda i: (i, 0)),
            pl.BlockSpec((be, 3), lambda i: (i, 0)),
            _const2((3, _NH)), _const2((1, _NH)), _SMEM_SPEC,
            _const2((_NH, _NH)), _const2((1, _NH)),
        ],
        out_specs=pl.BlockSpec((be, _NH), lambda i: (i, 0)),
        out_shape=jax.ShapeDtypeStruct((_E, _NH), jnp.float32),
    )(ev, es, p["l1"]["w"], p["l1"]["b"].reshape(1, -1), p["a"].reshape(1, 1),
      p["l2"]["w"], p["l2"]["b"].reshape(1, -1))


def _edge_mlp(ar, ac, eh, p):
    be = 4000
    ne_edges = ar.shape[0]
    w1 = p["l1"]["w"]
    return pl.pallas_call(
        _edge_mlp_body,
        grid=(ne_edges // be,),
        in_specs=[
            pl.BlockSpec((be, _NH), lambda i: (i, 0)),
            pl.BlockSpec((be, _NH), lambda i: (i, 0)),
            pl.BlockSpec((be, _NH), lambda i: (i, 0)),
            _const2((_NH, 128)), _const2((_NH, 128)), _const2((_NH, 128)),
            _const2((1, 128)), _const2((1, 128)), _const2((1, 128)), _SMEM_SPEC,
            _const2((128, _NH)), _const2((1, _NH)),
        ],
        out_specs=[
            pl.BlockSpec((be, _NH), lambda i: (i, 0)),
            pl.BlockSpec((be, _NH), lambda i: (i, 0)),
        ],
        out_shape=[
            jax.ShapeDtypeStruct((ne_edges, _NH), jnp.float32),
            jax.ShapeDtypeStruct((ne_edges, _NH), jnp.float32),
        ],
    )(ar, ac, eh, w1[:_NH], w1[_NH:2 * _NH], w1[2 * _NH:],
      p["l1"]["b"].reshape(1, -1), p["g"].reshape(1, -1),
      p["bt"].reshape(1, -1), p["a"].reshape(1, 1),
      p["l2"]["w"], p["l2"]["b"].reshape(1, -1))


def _edge_mlp1(ev, es, ar, ac, pe, p):
    be = 4000
    ne_edges = ar.shape[0]
    w1 = p["l1"]["w"]
    return pl.pallas_call(
        _edge_mlp1_body,
        grid=(ne_edges // be,),
        in_specs=[
            pl.BlockSpec((be, 3), lambda i: (i, 0)),
            pl.BlockSpec((be, 3), lambda i: (i, 0)),
            pl.BlockSpec((be, _NH), lambda i: (i, 0)),
            pl.BlockSpec((be, _NH), lambda i: (i, 0)),
            _const2((3, _NH)), _const2((1, _NH)), _SMEM_SPEC,
            _const2((_NH, _NH)), _const2((1, _NH)),
            _const2((_NH, 128)), _const2((_NH, 128)), _const2((_NH, 128)),
            _const2((1, 128)), _const2((1, 128)), _const2((1, 128)), _SMEM_SPEC,
            _const2((128, _NH)), _const2((1, _NH)),
        ],
        out_specs=[
            pl.BlockSpec((be, _NH), lambda i: (i, 0)),
            pl.BlockSpec((be, _NH), lambda i: (i, 0)),
        ],
        out_shape=[
            jax.ShapeDtypeStruct((ne_edges, _NH), jnp.float32),
            jax.ShapeDtypeStruct((ne_edges, _NH), jnp.float32),
        ],
    )(ev, es, ar, ac,
      pe["l1"]["w"], pe["l1"]["b"].reshape(1, -1), pe["a"].reshape(1, 1),
      pe["l2"]["w"], pe["l2"]["b"].reshape(1, -1),
      w1[:_NH], w1[_NH:2 * _NH], w1[2 * _NH:],
      p["l1"]["b"].reshape(1, -1), p["g"].reshape(1, -1),
      p["bt"].reshape(1, -1), p["a"].reshape(1, 1),
      p["l2"]["w"], p["l2"]["b"].reshape(1, -1))


def _node_mlp(xh, agg, p):
    bn = 5000
    w1 = p["l1"]["w"]
    return pl.pallas_call(
        _node_mlp_body,
        grid=(_N // bn,),
        in_specs=[
            pl.BlockSpec((bn, _NH), lambda i: (i, 0)),
            pl.BlockSpec((bn, _NH), lambda i: (i, 0)),
            _const2((_NH, 128)), _const2((_NH, 128)),
            _const2((1, 128)), _const2((1, 128)), _const2((1, 128)), _SMEM_SPEC,
            _const2((128, _NH)), _const2((1, _NH)),
        ],
        out_specs=pl.BlockSpec((bn, _NH), lambda i: (i, 0)),
        out_shape=jax.ShapeDtypeStruct((_N, _NH), jnp.float32),
    )(xh, agg, w1[:_NH], w1[_NH:],
      p["l1"]["b"].reshape(1, -1), p["g"].reshape(1, -1),
      p["bt"].reshape(1, -1), p["a"].reshape(1, 1),
      p["l2"]["w"], p["l2"]["b"].reshape(1, -1))


def _out_mlp(xh, batch2, p):
    bn = 5000
    return pl.pallas_call(
        _out_body,
        grid=(_N // bn,),
        in_specs=[
            pl.BlockSpec((bn, _NH), lambda i: (i, 0)),
            pl.BlockSpec((bn, 1), lambda i: (i, 0)),
            _const2((_NH, _NH)), _const2((1, _NH)), _const2((1, _NH)),
            _const2((1, _NH)), _SMEM_SPEC,
            _const2((_NH, _DOS)), _const2((1, _DOS)),
        ],
        out_specs=pl.BlockSpec((_NG, _DOS), lambda i: (0, 0)),
        out_shape=jax.ShapeDtypeStruct((_NG, _DOS), jnp.float32),
        compiler_params=pltpu.CompilerParams(
            dimension_semantics=("arbitrary",)),
    )(xh, batch2, p["l1"]["w"], p["l1"]["b"].reshape(1, -1),
      p["g"].reshape(1, -1), p["bt"].reshape(1, -1), p["a"].reshape(1, 1),
      p["l2"]["w"], p["l2"]["b"].reshape(1, -1))


def _make_gather(ne_edges, gk, gnb, groups, gtail):
    ept = ne_edges // _NW

    def body(xh_hbm, row_hbm, col_hbm, outr_hbm, outc_hbm,
             idx_v, rows_v, tail_v, gsems, wsems):
        c = lax.axis_index("c")
        s = lax.axis_index("s")
        wid = s * _NC + c
        base = wid * ept
        for idx_hbm, out_hbm in ((row_hbm, outr_hbm), (col_hbm, outc_hbm)):
            pltpu.sync_copy(idx_hbm.at[pl.ds(base, ept)], idx_v)

            def _group(g, carry, out_hbm=out_hbm):
                gcps = []
                for b in range(gnb):
                    j = g * gnb + b

                    @pl.when(g > 0)
                    def _(b=b, out_hbm=out_hbm):
                        pltpu.make_async_copy(
                            rows_v.at[b], out_hbm.at[pl.ds(0, gk)],
                            wsems.at[b]).wait()

                    gcps.append(pltpu.async_copy(
                        xh_hbm.at[idx_v.at[pl.ds(j * gk, gk)]],
                        rows_v.at[b], gsems.at[b]))
                for b in range(gnb):
                    j = g * gnb + b
                    gcps[b].wait()
                    pltpu.async_copy(rows_v.at[b],
                                     out_hbm.at[pl.ds(base + j * gk, gk)],
                                     wsems.at[b])
                return carry

            lax.fori_loop(0, groups, _group, 0)
            for b in range(gnb):
                pltpu.make_async_copy(rows_v.at[b], out_hbm.at[pl.ds(0, gk)],
                                      wsems.at[b]).wait()
            if gtail:
                toff = groups * gnb * gk
                pltpu.async_copy(xh_hbm.at[idx_v.at[pl.ds(toff, gtail)]],
                                 tail_v, gsems.at[0]).wait()
                pltpu.sync_copy(tail_v, out_hbm.at[pl.ds(base + toff, gtail)])

    return pl.kernel(
        body,
        out_type=[jax.ShapeDtypeStruct((ne_edges, _NH), jnp.float32),
                  jax.ShapeDtypeStruct((ne_edges, _NH), jnp.float32)],
        mesh=plsc.VectorSubcoreMesh(core_axis_name="c", subcore_axis_name="s"),
        scratch_types=[
            pltpu.VMEM((ept,), jnp.int32),
            pltpu.VMEM((gnb, gk, _NH), jnp.float32),
            pltpu.VMEM((max(gtail, 8), _NH), jnp.float32),
            pltpu.SemaphoreType.DMA((gnb,)),
            pltpu.SemaphoreType.DMA((gnb,)),
        ],
        compiler_params=pltpu.CompilerParams(use_tc_tiling_on_sc=False),
    )


_EHA = 416000
_EHB = _E - _EHA
_gather_calls = {}


def _gather(xh, row, col):
    n = row.shape[0]
    if n not in _gather_calls:
        ept = n // _NW
        full = ept // 96
        gnb = 5
        groups = full // gnb
        tail = ept - groups * gnb * 96
        _gather_calls[n] = _make_gather(n, 96, gnb, groups, tail)
    return _gather_calls[n](xh, row, col)


_F = _NH // _NC
_RPT = _N // _NS
_ZR = 256
_ZCOPIES = _RPT // _ZR

_SK = 80
_SNB = 8


def _make_scatter2():
    def body(neA_hbm, col2A_hbm, neB_hbm, col2B_hbm, out_hbm,
             colring, rows_v, zrow, shared, rsems, csems, ssems):
        c = lax.axis_index("c")
        s = lax.axis_index("s")
        foff = c * _F

        def _zfill(i, carry):
            zrow[i, pl.ds(0, 16)] = jnp.zeros((16,), jnp.float32)
            zrow[i, pl.ds(16, 16)] = jnp.zeros((16,), jnp.float32)
            return carry

        lax.fori_loop(0, _ZR, _zfill, 0)
        rbase = s * _RPT
        zcps = []
        for k in range(_ZCOPIES):
            zcps.append(pltpu.async_copy(
                zrow, shared.at[pl.ds(rbase + k * _ZR, _ZR)],
                rsems.at[k % _SNB]))
        zcps.append(pltpu.async_copy(
            zrow.at[pl.ds(0, _RPT - _ZCOPIES * _ZR)],
            shared.at[pl.ds(rbase + _ZCOPIES * _ZR, _RPT - _ZCOPIES * _ZR)],
            rsems.at[_ZCOPIES % _SNB]))
        for cp in zcps:
            cp.wait()
        plsc.subcore_barrier()

        for ne_hbm, col2_hbm, ne_edges in ((neA_hbm, col2A_hbm, _EHA),
                                           (neB_hbm, col2B_hbm, _EHB)):
            sept = ne_edges // _NS
            sch = sept // _SK
            sg = sch // _SNB
            ntail = sch - sg * _SNB
            ebase = s * sept
            cbase = s * sch

            def _group(g, carry, ne_hbm=ne_hbm, col2_hbm=col2_hbm,
                       ebase=ebase, cbase=cbase):
                cps = []
                for b in range(_SNB):
                    j = g * _SNB + b

                    @pl.when(g > 0)
                    def _(b=b):
                        pltpu.make_async_copy(
                            rows_v.at[b], shared.at[pl.ds(0, _SK)],
                            ssems.at[b]).wait()

                    cps.append((
                        pltpu.async_copy(col2_hbm.at[cbase + j],
                                         colring.at[b], csems.at[b]),
                        pltpu.async_copy(
                            ne_hbm.at[pl.ds(ebase + j * _SK, _SK),
                                      pl.ds(foff, _F)],
                            rows_v.at[b], rsems.at[b])))
                for b in range(_SNB):
                    ccp, rcp = cps[b]
                    ccp.wait()
                    rcp.wait()
                    pltpu.async_copy(rows_v.at[b], shared.at[colring.at[b]],
                                     ssems.at[b], add=True)
                return carry

            lax.fori_loop(0, sg, _group, 0)
            for b in range(_SNB):
                pltpu.make_async_copy(rows_v.at[b], shared.at[pl.ds(0, _SK)],
                                      ssems.at[b]).wait()
            for t in range(ntail):
                jt = sg * _SNB + t
                pltpu.sync_copy(col2_hbm.at[cbase + jt], colring.at[0])
                pltpu.sync_copy(
                    ne_hbm.at[pl.ds(ebase + jt * _SK, _SK), pl.ds(foff, _F)],
                    rows_v.at[0])
                pltpu.sync_copy(rows_v.at[0], shared.at[colring.at[0]],
                                add=True)
        plsc.subcore_barrier()
        pltpu.sync_copy(shared.at[pl.ds(rbase, _RPT)],
                        out_hbm.at[pl.ds(rbase, _RPT), pl.ds(foff, _F)])

    return pl.kernel(
        body,
        out_type=jax.ShapeDtypeStruct((_N, _NH), jnp.float32),
        mesh=plsc.VectorSubcoreMesh(core_axis_name="c",
                                    subcore_axis_name="s"),
        scratch_types=[
            pltpu.VMEM((_SNB, _SK), jnp.int32),
            pltpu.VMEM((_SNB, _SK, _F), jnp.float32),
            pltpu.VMEM((_ZR, _F), jnp.float32),
            pltpu.VMEM_SHARED((_N, _F), jnp.float32),
            pltpu.SemaphoreType.DMA((_SNB,)),
            pltpu.SemaphoreType.DMA((_SNB,)),
            pltpu.SemaphoreType.DMA((_SNB,)),
        ],
        compiler_params=pltpu.CompilerParams(use_tc_tiling_on_sc=False),
    )


_scatter2_call = None


def _scatter2(neA, col2A, neB, col2B):
    global _scatter2_call
    if _scatter2_call is None:
        _scatter2_call = _make_scatter2()
    return _scatter2_call(neA, col2A, neB, col2B)


def kernel(x, edge_vec, edge_shift, edge_index, batch, params):
    row = edge_index[0]
    col = edge_index[1]
    batch2 = batch.reshape(_N, 1)
    rows = (row[:_EHA], row[_EHA:])
    cols = (col[:_EHA], col[_EHA:])
    col2s = tuple(c.reshape(-1, _SK) for c in cols)
    evs = (edge_vec[:_EHA], edge_vec[_EHA:])
    ess = (edge_shift[:_EHA], edge_shift[_EHA:])

    xh = _enc_node(x, params["enc_node"])
    ehs = [None, None]
    for li, pr in enumerate(params["procs"]):
        xrA, xcA = _gather(xh, rows[0], cols[0])
        rB, cB, _ = lax.optimization_barrier((rows[1], cols[1], xrA))
        xrB, xcB = _gather(xh, rB, cB)
        gathered = [(xrA, xcA), (xrB, xcB)]
        nes = [None, None]
        for h in range(2):
            xr, xc = gathered[h]
            if li == 0:
                nes[h], ehs[h] = _edge_mlp1(evs[h], ess[h], xr, xc,
                                            params["enc_edge"], pr["edge"])
            else:
                nes[h], ehs[h] = _edge_mlp(xr, xc, ehs[h], pr["edge"])
        agg = _scatter2(nes[0], col2s[0], nes[1], col2s[1])
        xh = _node_mlp(xh, agg, pr["node"])
    return _out_mlp(xh, batch2, params["out"])


# speedup vs baseline: 1.0119x; 1.0119x over previous
import jax
import jax.numpy as jnp
from jax import lax
from jax.experimental import pallas as pl
from jax.experimental.pallas import tpu as pltpu
from jax.experimental.pallas import tpu_sc as plsc

_N = 50000
_E = 800000
_NH = 64
_NG = 64
_DOS = 51

_NC = 2
_NS = 16
_NW = _NC * _NS


def _dot(a, b):
    return jnp.dot(a, b, preferred_element_type=jnp.float32)


def _ln(h, g, bt):
    m = jnp.mean(h, axis=-1, keepdims=True)
    v = jnp.mean(h * h, axis=-1, keepdims=True) - m * m
    return (h - m) * lax.rsqrt(v + 1e-5) * g + bt


def _prelu(h, a):
    return jnp.where(h >= 0, h, a * h)


def _enc_node_body(x_ref, w1_ref, b1_ref, a_ref, w2_ref, b2_ref, o_ref):
    h = _dot(x_ref[...], w1_ref[...]) + b1_ref[...]
    h = _prelu(h, a_ref[0, 0])
    o_ref[...] = _dot(h, w2_ref[...]) + b2_ref[...]


def _enc_edge_body(ev_ref, es_ref, w1_ref, b1_ref, a_ref, w2_ref, b2_ref, o_ref):
    ev = ev_ref[...]
    el = jnp.sqrt(jnp.sum(ev * ev, axis=-1, keepdims=True))
    u = 2.0 * (el / 4.0 - 1.0)
    y = (1.0 - jnp.cos(jnp.pi * u)) / 2.0
    y = jnp.where(u > 0, 0.0, y)
    y = jnp.where(u < -1.0, 1.0, y)
    attr = y * es_ref[...]
    h = _dot(attr, w1_ref[...]) + b1_ref[...]
    h = _prelu(h, a_ref[0, 0])
    o_ref[...] = _dot(h, w2_ref[...]) + b2_ref[...]


def _edge_mlp_body(ar_ref, ac_ref, eh_ref, w1a_ref, w1b_ref, w1c_ref, b1_ref,
                   g_ref, bt_ref, a_ref, w2_ref, b2_ref, ne_ref, eo_ref):
    eh = eh_ref[...]
    h = (_dot(ar_ref[...], w1a_ref[...]) + _dot(ac_ref[...], w1b_ref[...])
         + _dot(eh, w1c_ref[...]) + b1_ref[...])
    h = _prelu(_ln(h, g_ref[...], bt_ref[...]), a_ref[0, 0])
    ne = _dot(h, w2_ref[...]) + b2_ref[...]
    ne_ref[...] = ne
    eo_ref[...] = eh + ne


def _edge_mlp1_body(ev_ref, es_ref, ar_ref, ac_ref,
                    ew1_ref, eb1_ref, ea_ref, ew2_ref, eb2_ref,
                    w1a_ref, w1b_ref, w1c_ref, b1_ref,
                    g_ref, bt_ref, a_ref, w2_ref, b2_ref, ne_ref, eo_ref):
    ev = ev_ref[...]
    el = jnp.sqrt(jnp.sum(ev * ev, axis=-1, keepdims=True))
    u = 2.0 * (el / 4.0 - 1.0)
    y = (1.0 - jnp.cos(jnp.pi * u)) / 2.0
    y = jnp.where(u > 0, 0.0, y)
    y = jnp.where(u < -1.0, 1.0, y)
    attr = y * es_ref[...]
    eh = _dot(attr, ew1_ref[...]) + eb1_ref[...]
    eh = _prelu(eh, ea_ref[0, 0])
    eh = _dot(eh, ew2_ref[...]) + eb2_ref[...]
    h = (_dot(ar_ref[...], w1a_ref[...]) + _dot(ac_ref[...], w1b_ref[...])
         + _dot(eh, w1c_ref[...]) + b1_ref[...])
    h = _prelu(_ln(h, g_ref[...], bt_ref[...]), a_ref[0, 0])
    ne = _dot(h, w2_ref[...]) + b2_ref[...]
    ne_ref[...] = ne
    eo_ref[...] = eh + ne


def _node_mlp_body(xh_ref, agga_ref, aggb_ref, w1a_ref, w1b_ref, b1_ref,
                   g_ref, bt_ref, a_ref, w2_ref, b2_ref, o_ref):
    agg = agga_ref[...] + aggb_ref[...]
    h = (_dot(xh_ref[...], w1a_ref[...]) + _dot(agg, w1b_ref[...])
         + b1_ref[...])
    h = _prelu(_ln(h, g_ref[...], bt_ref[...]), a_ref[0, 0])
    o_ref[...] = xh_ref[...] + _dot(h, w2_ref[...]) + b2_ref[...]


def _out_body(xh_ref, b_ref, w1_ref, b1_ref, g_ref, bt_ref, a_ref, w2_ref,
              b2_ref, o_ref):
    i = pl.program_id(0)
    h = _dot(xh_ref[...], w1_ref[...]) + b1_ref[...]
    h = _prelu(_ln(h, g_ref[...], bt_ref[...]), a_ref[0, 0])
    dos = _dot(h, w2_ref[...]) + b2_ref[...]
    oh = (b_ref[...] == lax.broadcasted_iota(jnp.int32, (1, _NG), 1))
    oh = oh.astype(jnp.float32)
    contrib = lax.dot_general(oh, dos, (((0,), (0,)), ((), ())),
                              preferred_element_type=jnp.float32)

    @pl.when(i == 0)
    def _():
        o_ref[...] = jnp.zeros_like(o_ref)

    o_ref[...] += contrib


def _const2(shape):
    return pl.BlockSpec(shape, lambda i: (0, 0))


_SMEM_SPEC = pl.BlockSpec(memory_space=pltpu.SMEM)


def _enc_node(x, p):
    bn = 5000
    return pl.pallas_call(
        _enc_node_body,
        grid=(_N // bn,),
        in_specs=[
            pl.BlockSpec((bn, 128), lambda i: (i, 0)),
            _const2((128, _NH)), _const2((1, _NH)), _SMEM_SPEC,
            _const2((_NH, _NH)), _const2((1, _NH)),
        ],
        out_specs=pl.BlockSpec((bn, _NH), lambda i: (i, 0)),
        out_shape=jax.ShapeDtypeStruct((_N, _NH), jnp.float32),
    )(x, p["l1"]["w"], p["l1"]["b"].reshape(1, -1), p["a"].reshape(1, 1),
      p["l2"]["w"], p["l2"]["b"].reshape(1, -1))


def _enc_edge(ev, es, p):
    be = 8000
    return pl.pallas_call(
        _enc_edge_body,
        grid=(_E // be,),
        in_specs=[
            pl.BlockSpec((be, 3), lambda i: (i, 0)),
            pl.BlockSpec((be, 3), lambda i: (i, 0)),
            _const2((3, _NH)), _const2((1, _NH)), _SMEM_SPEC,
            _const2((_NH, _NH)), _const2((1, _NH)),
        ],
        out_specs=pl.BlockSpec((be, _NH), lambda i: (i, 0)),
        out_shape=jax.ShapeDtypeStruct((_E, _NH), jnp.float32),
    )(ev, es, p["l1"]["w"], p["l1"]["b"].reshape(1, -1), p["a"].reshape(1, 1),
      p["l2"]["w"], p["l2"]["b"].reshape(1, -1))


def _edge_mlp(ar, ac, eh, p):
    be = 4000
    ne_edges = ar.shape[0]
    w1 = p["l1"]["w"]
    return pl.pallas_call(
        _edge_mlp_body,
        grid=(ne_edges // be,),
        in_specs=[
            pl.BlockSpec((be, _NH), lambda i: (i, 0)),
            pl.BlockSpec((be, _NH), lambda i: (i, 0)),
            pl.BlockSpec((be, _NH), lambda i: (i, 0)),
            _const2((_NH, 128)), _const2((_NH, 128)), _const2((_NH, 128)),
            _const2((1, 128)), _const2((1, 128)), _const2((1, 128)), _SMEM_SPEC,
            _const2((128, _NH)), _const2((1, _NH)),
        ],
        out_specs=[
            pl.BlockSpec((be, _NH), lambda i: (i, 0)),
            pl.BlockSpec((be, _NH), lambda i: (i, 0)),
        ],
        out_shape=[
            jax.ShapeDtypeStruct((ne_edges, _NH), jnp.float32),
            jax.ShapeDtypeStruct((ne_edges, _NH), jnp.float32),
        ],
    )(ar, ac, eh, w1[:_NH], w1[_NH:2 * _NH], w1[2 * _NH:],
      p["l1"]["b"].reshape(1, -1), p["g"].reshape(1, -1),
      p["bt"].reshape(1, -1), p["a"].reshape(1, 1),
      p["l2"]["w"], p["l2"]["b"].reshape(1, -1))


def _edge_mlp1(ev, es, ar, ac, pe, p):
    be = 4000
    ne_edges = ar.shape[0]
    w1 = p["l1"]["w"]
    return pl.pallas_call(
        _edge_mlp1_body,
        grid=(ne_edges // be,),
        in_specs=[
            pl.BlockSpec((be, 3), lambda i: (i, 0)),
            pl.BlockSpec((be, 3), lambda i: (i, 0)),
            pl.BlockSpec((be, _NH), lambda i: (i, 0)),
            pl.BlockSpec((be, _NH), lambda i: (i, 0)),
            _const2((3, _NH)), _const2((1, _NH)), _SMEM_SPEC,
            _const2((_NH, _NH)), _const2((1, _NH)),
            _const2((_NH, 128)), _const2((_NH, 128)), _const2((_NH, 128)),
            _const2((1, 128)), _const2((1, 128)), _const2((1, 128)), _SMEM_SPEC,
            _const2((128, _NH)), _const2((1, _NH)),
        ],
        out_specs=[
            pl.BlockSpec((be, _NH), lambda i: (i, 0)),
            pl.BlockSpec((be, _NH), lambda i: (i, 0)),
        ],
        out_shape=[
            jax.ShapeDtypeStruct((ne_edges, _NH), jnp.float32),
            jax.ShapeDtypeStruct((ne_edges, _NH), jnp.float32),
        ],
    )(ev, es, ar, ac,
      pe["l1"]["w"], pe["l1"]["b"].reshape(1, -1), pe["a"].reshape(1, 1),
      pe["l2"]["w"], pe["l2"]["b"].reshape(1, -1),
      w1[:_NH], w1[_NH:2 * _NH], w1[2 * _NH:],
      p["l1"]["b"].reshape(1, -1), p["g"].reshape(1, -1),
      p["bt"].reshape(1, -1), p["a"].reshape(1, 1),
      p["l2"]["w"], p["l2"]["b"].reshape(1, -1))


def _node_mlp(xh, agga, aggb, p):
    bn = 5000
    w1 = p["l1"]["w"]
    return pl.pallas_call(
        _node_mlp_body,
        grid=(_N // bn,),
        in_specs=[
            pl.BlockSpec((bn, _NH), lambda i: (i, 0)),
            pl.BlockSpec((bn, _NH), lambda i: (i, 0)),
            pl.BlockSpec((bn, _NH), lambda i: (i, 0)),
            _const2((_NH, 128)), _const2((_NH, 128)),
            _const2((1, 128)), _const2((1, 128)), _const2((1, 128)), _SMEM_SPEC,
            _const2((128, _NH)), _const2((1, _NH)),
        ],
        out_specs=pl.BlockSpec((bn, _NH), lambda i: (i, 0)),
        out_shape=jax.ShapeDtypeStruct((_N, _NH), jnp.float32),
    )(xh, agga, aggb, w1[:_NH], w1[_NH:],
      p["l1"]["b"].reshape(1, -1), p["g"].reshape(1, -1),
      p["bt"].reshape(1, -1), p["a"].reshape(1, 1),
      p["l2"]["w"], p["l2"]["b"].reshape(1, -1))


def _out_mlp(xh, batch2, p):
    bn = 5000
    return pl.pallas_call(
        _out_body,
        grid=(_N // bn,),
        in_specs=[
            pl.BlockSpec((bn, _NH), lambda i: (i, 0)),
            pl.BlockSpec((bn, 1), lambda i: (i, 0)),
            _const2((_NH, _NH)), _const2((1, _NH)), _const2((1, _NH)),
            _const2((1, _NH)), _SMEM_SPEC,
            _const2((_NH, _DOS)), _const2((1, _DOS)),
        ],
        out_specs=pl.BlockSpec((_NG, _DOS), lambda i: (0, 0)),
        out_shape=jax.ShapeDtypeStruct((_NG, _DOS), jnp.float32),
        compiler_params=pltpu.CompilerParams(
            dimension_semantics=("arbitrary",)),
    )(xh, batch2, p["l1"]["w"], p["l1"]["b"].reshape(1, -1),
      p["g"].reshape(1, -1), p["bt"].reshape(1, -1), p["a"].reshape(1, 1),
      p["l2"]["w"], p["l2"]["b"].reshape(1, -1))


def _make_gather(ne_edges, gk, gnb, groups, gtail):
    ept = ne_edges // _NW

    def body(xh_hbm, row_hbm, col_hbm, outr_hbm, outc_hbm,
             idx_v, rows_v, tail_v, gsems, wsems):
        c = lax.axis_index("c")
        s = lax.axis_index("s")
        wid = s * _NC + c
        base = wid * ept
        for idx_hbm, out_hbm in ((row_hbm, outr_hbm), (col_hbm, outc_hbm)):
            pltpu.sync_copy(idx_hbm.at[pl.ds(base, ept)], idx_v)

            def _group(g, carry, out_hbm=out_hbm):
                gcps = []
                for b in range(gnb):
                    j = g * gnb + b

                    @pl.when(g > 0)
                    def _(b=b, out_hbm=out_hbm):
                        pltpu.make_async_copy(
                            rows_v.at[b], out_hbm.at[pl.ds(0, gk)],
                            wsems.at[b]).wait()

                    gcps.append(pltpu.async_copy(
                        xh_hbm.at[idx_v.at[pl.ds(j * gk, gk)]],
                        rows_v.at[b], gsems.at[b]))
                for b in range(gnb):
                    j = g * gnb + b
                    gcps[b].wait()
                    pltpu.async_copy(rows_v.at[b],
                                     out_hbm.at[pl.ds(base + j * gk, gk)],
                                     wsems.at[b])
                return carry

            lax.fori_loop(0, groups, _group, 0)
            for b in range(gnb):
                pltpu.make_async_copy(rows_v.at[b], out_hbm.at[pl.ds(0, gk)],
                                      wsems.at[b]).wait()
            if gtail:
                toff = groups * gnb * gk
                pltpu.async_copy(xh_hbm.at[idx_v.at[pl.ds(toff, gtail)]],
                                 tail_v, gsems.at[0]).wait()
                pltpu.sync_copy(tail_v, out_hbm.at[pl.ds(base + toff, gtail)])

    return pl.kernel(
        body,
        out_type=[jax.ShapeDtypeStruct((ne_edges, _NH), jnp.float32),
                  jax.ShapeDtypeStruct((ne_edges, _NH), jnp.float32)],
        mesh=plsc.VectorSubcoreMesh(core_axis_name="c", subcore_axis_name="s"),
        scratch_types=[
            pltpu.VMEM((ept,), jnp.int32),
            pltpu.VMEM((gnb, gk, _NH), jnp.float32),
            pltpu.VMEM((max(gtail, 8), _NH), jnp.float32),
            pltpu.SemaphoreType.DMA((gnb,)),
            pltpu.SemaphoreType.DMA((gnb,)),
        ],
        compiler_params=pltpu.CompilerParams(use_tc_tiling_on_sc=False),
    )


_EHA = 416000
_EHB = _E - _EHA
_gather_calls = {}


def _gather(xh, row, col):
    n = row.shape[0]
    if n not in _gather_calls:
        ept = n // _NW
        full = ept // 96
        gnb = 5
        groups = full // gnb
        tail = ept - groups * gnb * 96
        _gather_calls[n] = _make_gather(n, 96, gnb, groups, tail)
    return _gather_calls[n](xh, row, col)


_F = _NH // _NC
_RPT = _N // _NS
_ZR = 256
_ZCOPIES = _RPT // _ZR

_SK = 80
_SNB = 8


def _make_scatter(ne_edges):
    sept = ne_edges // _NS
    sch = sept // _SK
    sg = sch // _SNB
    ntail = sch - sg * _SNB

    def body(ne_hbm, col2_hbm, out_hbm, colring, rows_v, zrow,
             shared, rsems, csems, ssems):
        c = lax.axis_index("c")
        s = lax.axis_index("s")
        foff = c * _F

        def _zfill(i, carry):
            zrow[i, pl.ds(0, 16)] = jnp.zeros((16,), jnp.float32)
            zrow[i, pl.ds(16, 16)] = jnp.zeros((16,), jnp.float32)
            return carry

        lax.fori_loop(0, _ZR, _zfill, 0)
        rbase = s * _RPT
        zcps = []
        for k in range(_ZCOPIES):
            zcps.append(pltpu.async_copy(
                zrow, shared.at[pl.ds(rbase + k * _ZR, _ZR)],
                rsems.at[k % _SNB]))
        zcps.append(pltpu.async_copy(
            zrow.at[pl.ds(0, _RPT - _ZCOPIES * _ZR)],
            shared.at[pl.ds(rbase + _ZCOPIES * _ZR, _RPT - _ZCOPIES * _ZR)],
            rsems.at[_ZCOPIES % _SNB]))
        for cp in zcps:
            cp.wait()
        plsc.subcore_barrier()

        ebase = s * sept
        cbase = s * sch

        def _group(g, carry):
            cps = []
            for b in range(_SNB):
                j = g * _SNB + b

                @pl.when(g > 0)
                def _(b=b):
                    pltpu.make_async_copy(
                        rows_v.at[b], shared.at[pl.ds(0, _SK)],
                        ssems.at[b]).wait()

                cps.append((
                    pltpu.async_copy(col2_hbm.at[cbase + j], colring.at[b],
                                     csems.at[b]),
                    pltpu.async_copy(
                        ne_hbm.at[pl.ds(ebase + j * _SK, _SK),
                                  pl.ds(foff, _F)],
                        rows_v.at[b], rsems.at[b])))
            for b in range(_SNB):
                ccp, rcp = cps[b]
                ccp.wait()
                rcp.wait()
                pltpu.async_copy(rows_v.at[b], shared.at[colring.at[b]],
                                 ssems.at[b], add=True)
            return carry

        lax.fori_loop(0, sg, _group, 0)
        for b in range(_SNB):
            pltpu.make_async_copy(rows_v.at[b], shared.at[pl.ds(0, _SK)],
                                  ssems.at[b]).wait()
        for t in range(ntail):
            jt = sg * _SNB + t
            pltpu.sync_copy(col2_hbm.at[cbase + jt], colring.at[0])
            pltpu.sync_copy(
                ne_hbm.at[pl.ds(ebase + jt * _SK, _SK), pl.ds(foff, _F)],
                rows_v.at[0])
            pltpu.sync_copy(rows_v.at[0], shared.at[colring.at[0]], add=True)
        plsc.subcore_barrier()
        pltpu.sync_copy(shared.at[pl.ds(rbase, _RPT)],
                        out_hbm.at[pl.ds(rbase, _RPT), pl.ds(foff, _F)])

    return pl.kernel(
        body,
        out_type=jax.ShapeDtypeStruct((_N, _NH), jnp.float32),
        mesh=plsc.VectorSubcoreMesh(core_axis_name="c",
                                    subcore_axis_name="s"),
        scratch_types=[
            pltpu.VMEM((_SNB, _SK), jnp.int32),
            pltpu.VMEM((_SNB, _SK, _F), jnp.float32),
            pltpu.VMEM((_ZR, _F), jnp.float32),
            pltpu.VMEM_SHARED((_N, _F), jnp.float32),
            pltpu.SemaphoreType.DMA((_SNB,)),
            pltpu.SemaphoreType.DMA((_SNB,)),
            pltpu.SemaphoreType.DMA((_SNB,)),
        ],
        compiler_params=pltpu.CompilerParams(use_tc_tiling_on_sc=False),
    )


_scatter_calls = {}


def _scatter(ne, col2):
    n = ne.shape[0]
    if n not in _scatter_calls:
        _scatter_calls[n] = _make_scatter(n)
    return _scatter_calls[n](ne, col2)


def kernel(x, edge_vec, edge_shift, edge_index, batch, params):
    row = edge_index[0]
    col = edge_index[1]
    batch2 = batch.reshape(_N, 1)
    rows = (row[:_EHA], row[_EHA:])
    cols = (col[:_EHA], col[_EHA:])
    col2s = tuple(c.reshape(-1, _SK) for c in cols)
    evs = (edge_vec[:_EHA], edge_vec[_EHA:])
    ess = (edge_shift[:_EHA], edge_shift[_EHA:])

    xh = _enc_node(x, params["enc_node"])
    ehs = [None, None]
    for li, pr in enumerate(params["procs"]):
        xrA, xcA = _gather(xh, rows[0], cols[0])
        rB, cB, _ = lax.optimization_barrier((rows[1], cols[1], xrA))
        xrB, xcB = _gather(xh, rB, cB)
        gathered = [(xrA, xcA), (xrB, xcB)]
        nes = [None, None]
        for h in range(2):
            xr, xc = gathered[h]
            if li == 0:
                nes[h], ehs[h] = _edge_mlp1(evs[h], ess[h], xr, xc,
                                            params["enc_edge"], pr["edge"])
            else:
                nes[h], ehs[h] = _edge_mlp(xr, xc, ehs[h], pr["edge"])
        neA, _ = lax.optimization_barrier((nes[0], xrB))
        aggA = _scatter(neA, col2s[0])
        neB, _ = lax.optimization_barrier((nes[1], aggA))
        aggB = _scatter(neB, col2s[1])
        xh = _node_mlp(xh, aggA, aggB, pr["node"])
    return _out_mlp(xh, batch2, params["out"])
